# baseline (device time: 13321 ns/iter reference)
import jax
import jax.numpy as jnp
from jax import lax
from jax.experimental import pallas as pl
from jax.experimental.pallas import tpu as pltpu

N_X = 2
N_SPLIT = 2


def kernel(x):
    m_per, n_per = x.shape
    m_global = N_X * m_per
    n_half = n_per // N_SPLIT
    scale = 1.0 / m_global

    def body(x_ref, out_ref, acc_ref, comm_ref, send_sems, recv_sems):
        j = pl.program_id(0)
        my_x = lax.axis_index("x")
        my_y = lax.axis_index("y")
        nbr = (1 - my_x, my_y)
        barrier_sem = pltpu.get_barrier_semaphore()

        def make_rdma(slot):
            return pltpu.make_async_remote_copy(
                src_ref=acc_ref.at[slot],
                dst_ref=comm_ref.at[slot],
                send_sem=send_sems.at[slot],
                recv_sem=recv_sems.at[slot],
                device_id=nbr,
                device_id_type=pl.DeviceIdType.MESH,
            )

        @pl.when(j == 0)
        def _():
            pl.semaphore_signal(
                barrier_sem, inc=1, device_id=nbr,
                device_id_type=pl.DeviceIdType.MESH,
            )
            acc_ref[0] = jnp.sum(x_ref[...], axis=0, keepdims=True)
            pl.semaphore_wait(barrier_sem, 1)
            make_rdma(0).start()

        @pl.when(j == N_SPLIT - 1)
        def _():
            acc_ref[1] = jnp.sum(x_ref[...], axis=0, keepdims=True)
            make_rdma(1).start()
            r0 = make_rdma(0)
            r0.wait_recv()
            out_ref[:, :n_half] = (acc_ref[0] + comm_ref[0]) * scale
            r1 = make_rdma(1)
            r1.wait_recv()
            out_ref[:, n_half:] = (acc_ref[1] + comm_ref[1]) * scale
            r0.wait_send()
            r1.wait_send()

    return pl.pallas_call(
        body,
        grid=(N_SPLIT,),
        out_shape=jax.ShapeDtypeStruct((1, n_per), jnp.float32),
        in_specs=[
            pl.BlockSpec(
                (m_per, n_half), lambda j: (0, j), memory_space=pltpu.VMEM
            )
        ],
        out_specs=pl.BlockSpec(
            (1, n_per), lambda j: (0, 0), memory_space=pltpu.VMEM
        ),
        scratch_shapes=[
            pltpu.VMEM((N_SPLIT, 1, n_half), jnp.float32),
            pltpu.VMEM((N_SPLIT, 1, n_half), jnp.float32),
            pltpu.SemaphoreType.DMA((N_SPLIT,)),
            pltpu.SemaphoreType.DMA((N_SPLIT,)),
        ],
        compiler_params=pltpu.CompilerParams(
            collective_id=0,
            dimension_semantics=("arbitrary",),
        ),
    )(x)


# device time: 12018 ns/iter; 1.1084x vs baseline; 1.1084x over previous
import jax
import jax.numpy as jnp
from jax import lax
from jax.experimental import pallas as pl
from jax.experimental.pallas import tpu as pltpu

N_X = 2
N_BLOCKS = 8
N_MSG = 2


def kernel(x):
    m_per, n_per = x.shape
    m_global = N_X * m_per
    block_m = m_per // N_BLOCKS
    half = N_BLOCKS // N_MSG
    scale = 1.0 / m_global

    def body(x_hbm, out_ref, xbuf, acc_ref, comm_ref,
             copy_sems, send_sems, recv_sems):
        my_x = lax.axis_index("x")
        my_y = lax.axis_index("y")
        nbr = (1 - my_x, my_y)

        barrier_sem = pltpu.get_barrier_semaphore()
        pl.semaphore_signal(
            barrier_sem, inc=1, device_id=nbr,
            device_id_type=pl.DeviceIdType.MESH,
        )

        copies = []
        for b in range(N_BLOCKS):
            cp = pltpu.make_async_copy(
                x_hbm.at[pl.ds(b * block_m, block_m), :],
                xbuf.at[b],
                copy_sems.at[b],
            )
            cp.start()
            copies.append(cp)

        def make_rdma(slot):
            return pltpu.make_async_remote_copy(
                src_ref=acc_ref.at[slot],
                dst_ref=comm_ref.at[slot],
                send_sem=send_sems.at[slot],
                recv_sem=recv_sems.at[slot],
                device_id=nbr,
                device_id_type=pl.DeviceIdType.MESH,
            )

        for msg in range(N_MSG):
            copies[msg * half].wait()
            v = jnp.sum(xbuf[msg * half], axis=0, keepdims=True)
            for b in range(msg * half + 1, (msg + 1) * half):
                copies[b].wait()
                v += jnp.sum(xbuf[b], axis=0, keepdims=True)
            acc_ref[msg] = v
            if msg == 0:
                pl.semaphore_wait(barrier_sem, 1)
            make_rdma(msg).start()

        r0 = make_rdma(0)
        r1 = make_rdma(1)
        r0.wait_recv()
        r1.wait_recv()
        out_ref[...] = (
            (acc_ref[0] + acc_ref[1]) + (comm_ref[0] + comm_ref[1])
        ) * scale
        r0.wait_send()
        r1.wait_send()

    return pl.pallas_call(
        body,
        out_shape=jax.ShapeDtypeStruct((1, n_per), jnp.float32),
        in_specs=[pl.BlockSpec(memory_space=pl.ANY)],
        out_specs=pl.BlockSpec(memory_space=pltpu.VMEM),
        scratch_shapes=[
            pltpu.VMEM((N_BLOCKS, block_m, n_per), jnp.float32),
            pltpu.VMEM((N_MSG, 1, n_per), jnp.float32),
            pltpu.VMEM((N_MSG, 1, n_per), jnp.float32),
            pltpu.SemaphoreType.DMA((N_BLOCKS,)),
            pltpu.SemaphoreType.DMA((N_MSG,)),
            pltpu.SemaphoreType.DMA((N_MSG,)),
        ],
        compiler_params=pltpu.CompilerParams(collective_id=0),
    )(x)


# device time: 10734 ns/iter; 1.2410x vs baseline; 1.1196x over previous
import jax
import jax.numpy as jnp
from jax import lax
from jax.experimental import pallas as pl
from jax.experimental.pallas import tpu as pltpu

N_X = 2


def kernel(x):
    m_per, n_per = x.shape
    m_global = N_X * m_per
    scale = 1.0 / m_global

    def body(x_ref, out_ref, acc_ref, comm_ref, send_sem, recv_sem):
        my_x = lax.axis_index("x")
        my_y = lax.axis_index("y")
        nbr = (1 - my_x, my_y)

        barrier_sem = pltpu.get_barrier_semaphore()
        pl.semaphore_signal(
            barrier_sem, inc=1, device_id=nbr,
            device_id_type=pl.DeviceIdType.MESH,
        )

        acc_ref[...] = jnp.sum(x_ref[...], axis=0, keepdims=True)

        pl.semaphore_wait(barrier_sem, 1)
        rdma = pltpu.make_async_remote_copy(
            src_ref=acc_ref,
            dst_ref=comm_ref,
            send_sem=send_sem,
            recv_sem=recv_sem,
            device_id=nbr,
            device_id_type=pl.DeviceIdType.MESH,
        )
        rdma.start()
        rdma.wait_recv()
        out_ref[...] = (acc_ref[...] + comm_ref[...]) * scale
        rdma.wait_send()

    return pl.pallas_call(
        body,
        out_shape=jax.ShapeDtypeStruct((1, n_per), jnp.float32),
        in_specs=[pl.BlockSpec(memory_space=pltpu.VMEM)],
        out_specs=pl.BlockSpec(memory_space=pltpu.VMEM),
        scratch_shapes=[
            pltpu.VMEM((1, n_per), jnp.float32),
            pltpu.VMEM((1, n_per), jnp.float32),
            pltpu.SemaphoreType.DMA,
            pltpu.SemaphoreType.DMA,
        ],
        compiler_params=pltpu.CompilerParams(collective_id=0),
    )(x)
